# baseline (speedup 1.0000x reference)
import functools

import jax
import jax.numpy as jnp
from jax import lax
from jax.experimental import pallas as pl
from jax.experimental.pallas import tpu as pltpu
from jax.experimental.pallas import tpu_sc as plsc

H = 512
W = 512
NPH = 32
NPW = 32
PH = H // NPH
PW = W // NPW
NP = NPH * NPW
C = 192

CHUNK = PW * C
NCHUNKS = NP * PH
G = 16

_info = plsc.get_sparse_core_info()
NC, NS, L = _info.num_cores, _info.num_subcores, _info.num_lanes
NW = NC * NS
CPW = NCHUNKS // NW
NGROUPS = CPW // G


def _sc_body(x_hbm, out_hbm, idx0, idx1, buf0, buf1, sem0, sem1):
    wid = lax.axis_index("s") * NC + lax.axis_index("c")
    base = wid * CPW
    lane = lax.iota(jnp.int32, L)

    def chunk_idx(g):
        c = base + g * G + lane
        p = c >> 4
        i = c & 15
        ph = p >> 5
        pw = p & 31
        return ph * (NPH * PH) + i * NPW + pw

    def start(g, idx_ref, buf, sem):
        idx_ref[...] = chunk_idx(g)
        pltpu.async_copy(x_hbm.at[idx_ref], buf, sem)

    start(0, idx0, buf0, sem0)

    def body(k, _):
        g0 = 2 * k
        g1 = 2 * k + 1
        start(g1, idx1, buf1, sem1)
        pltpu.make_async_copy(x_hbm.at[idx0], buf0, sem0).wait()
        pltpu.sync_copy(buf0, out_hbm.at[pl.ds(base + g0 * G, G)])

        @pl.when(k < NGROUPS // 2 - 1)
        def _():
            start(g0 + 2, idx0, buf0, sem0)

        pltpu.make_async_copy(x_hbm.at[idx1], buf1, sem1).wait()
        pltpu.sync_copy(buf1, out_hbm.at[pl.ds(base + g1 * G, G)])
        return 0

    lax.fori_loop(0, NGROUPS // 2, body, 0)


@functools.partial(
    pl.kernel,
    out_type=jax.ShapeDtypeStruct((NCHUNKS, CHUNK), jnp.float32),
    mesh=plsc.VectorSubcoreMesh(core_axis_name="c", subcore_axis_name="s"),
    scratch_types=[
        pltpu.VMEM((L,), jnp.int32),
        pltpu.VMEM((L,), jnp.int32),
        pltpu.VMEM((G, CHUNK), jnp.float32),
        pltpu.VMEM((G, CHUNK), jnp.float32),
        pltpu.SemaphoreType.DMA,
        pltpu.SemaphoreType.DMA,
    ],
)
def _patch_gather(x_hbm, out_hbm, idx0, idx1, buf0, buf1, sem0, sem1):
    _sc_body(x_hbm, out_hbm, idx0, idx1, buf0, buf1, sem0, sem1)


def kernel(x, mesh_pos, batch_idx):
    del mesh_pos, batch_idx
    x2d = x.reshape(NCHUNKS, CHUNK)
    out2d = _patch_gather(x2d)
    return out2d.reshape(1, NP, PH, PW, C)
